# Initial kernel scaffold; baseline (speedup 1.0000x reference)
import jax
import jax.numpy as jnp
from jax.experimental import pallas as pl


def kernel(log_seqs, batch_user_list, item_emb, neg_bsk_idx, neg_items):
    raise NotImplementedError("write your pallas kernel here")


# speedup vs baseline: 1.3275x; 1.3275x over previous
import functools

import jax
import jax.numpy as jnp
from jax.experimental import pallas as pl
from jax.experimental.pallas import tpu as pltpu

NUM_ITEM = 50000
EMB = 128
NUM_LAYER = 2
PAD = 50000
B = 1024
NB = 8
BSK = 16
E = B * NB


def _loss_kernel(uloss_ref, pos_bsk_ref, neg_bsk_ref, pos_rep_ref, neg_rep_ref,
                 sumsq_ref, out_ref):
    u = uloss_ref[...]
    y_ui = jnp.sum(u * pos_bsk_ref[...], axis=-1)
    y_uj = jnp.sum(u * neg_bsk_ref[...], axis=-1)
    eps = 1e-08
    r = jax.nn.sigmoid(y_ui - y_uj)
    r = jnp.where(r == 0.0, eps, r)
    loss_1 = -jnp.mean(jnp.log(r))

    pos = pos_rep_ref[...].reshape(B, BSK, EMB)
    neg = neg_rep_ref[...].reshape(B, BSK, EMB)
    yui = jnp.sum(u[:, None, :] * pos, axis=-1)
    yuj = jnp.sum(u[:, None, :] * neg, axis=-1)
    r2 = jax.nn.sigmoid(yui - yuj)
    r2 = jnp.where(r2 == 0.0, eps, r2)
    loss_2 = jnp.mean(-jnp.mean(jnp.log(r2), axis=1))

    l2reg = 0.0001 * (sumsq_ref[0, 0] / 2.0)
    lane = jax.lax.broadcasted_iota(jnp.int32, (1, 128), 1)
    out_ref[...] = jnp.where(lane == 0, loss_1,
                             jnp.where(lane == 1, loss_2, l2reg))


def kernel(log_seqs, batch_user_list, item_emb, neg_bsk_idx, neg_items):
    N = NUM_ITEM + 1
    row = log_seqs.reshape(-1)
    col = jnp.repeat(jnp.arange(E), BSK)
    ones = jnp.ones(row.shape[0], jnp.float32)
    DV = jax.ops.segment_sum(ones, row, num_segments=N)
    dv2 = jnp.where(DV > 0, jax.lax.rsqrt(jnp.maximum(DV, 1.0)), 0.0)
    de2 = 1.0 / 4.0

    def G_mm(xb):
        return jax.ops.segment_sum((dv2[row] * de2)[:, None] * xb[col], row,
                                   num_segments=N)

    def BD_mm(xn):
        return jax.ops.segment_sum((dv2[row] * de2)[:, None] * xn[row], col,
                                   num_segments=E)

    x = item_emb
    item_reps = [x]
    basket_reps = []
    for _ in range(NUM_LAYER):
        bas = BD_mm(x)
        basket_reps.append(bas)
        x = G_mm(bas)
        item_reps.append(x)
    item_rep = (item_reps[0] + item_reps[1] + item_reps[2]) / 3.0
    basket_rep = (basket_reps[0] + basket_reps[1]) / 2.0

    br = basket_rep.reshape(B, NB, EMB)
    user_rep_for_loss = jnp.mean(br[:, : NB - 1], axis=1)
    pos_bsk = br[:, NB - 1]
    neg_bsk = basket_rep[neg_bsk_idx]

    pos_items = log_seqs[:, NB - 1]
    pos_rep = item_rep[pos_items].reshape(B * BSK, EMB)
    neg_rep = item_rep[neg_items].reshape(B * BSK, EMB)
    sumsq = jnp.sum(item_emb ** 2).reshape(1, 1)

    out = pl.pallas_call(
        _loss_kernel,
        out_shape=jax.ShapeDtypeStruct((1, 128), jnp.float32),
        in_specs=[
            pl.BlockSpec((B, EMB), lambda: (0, 0)),
            pl.BlockSpec((B, EMB), lambda: (0, 0)),
            pl.BlockSpec((B, EMB), lambda: (0, 0)),
            pl.BlockSpec((B * BSK, EMB), lambda: (0, 0)),
            pl.BlockSpec((B * BSK, EMB), lambda: (0, 0)),
            pl.BlockSpec((1, 1), lambda: (0, 0)),
        ],
        out_specs=pl.BlockSpec((1, 128), lambda: (0, 0)),
    )(user_rep_for_loss, pos_bsk, neg_bsk, pos_rep, neg_rep, sumsq)
    return (out[0, 0], out[0, 1], out[0, 2])


# speedup vs baseline: 2.7520x; 2.0730x over previous
import functools

import jax
import jax.numpy as jnp
from jax import lax
from jax.experimental import pallas as pl
from jax.experimental.pallas import tpu as pltpu
from jax.experimental.pallas import tpu_sc as plsc

NUM_ITEM = 50000
EMB = 128
PAD = 50000
B = 1024
NB = 8
BSK = 16
E = B * NB
NE = E * BSK
N = NUM_ITEM + 1
NPAD = 50176
NCHUNK = 8
C = NPAD // NCHUNK
CACC = C + 128
LCAP = 4096 + 128
NC = 2
NS = 16
NW = NC * NS
PER_TILE = NE // NW
HSLC = NPAD // NS
WB = C // NS
ZR = CACC // NS

_mesh = plsc.VectorSubcoreMesh(core_axis_name="c", subcore_axis_name="s")
_sc_params = pltpu.CompilerParams(needs_layout_passes=False)
_f32 = jnp.float32
_i32 = jnp.int32


def _zero_vmem(ref, n):
    z = jnp.zeros((16,), ref.dtype)

    def body(i, _):
        ref[pl.ds(i * 16, 16)] = z
        return 0

    lax.fori_loop(0, n // 16, body, 0)


def _k1_body(rows_hbm, dvp_hbm, lists_hbm, counts_hbm,
             idx_v, hist, b0, b1, b2, b3, b4, b5, b6, b7, cnt_v, sem):
    c = lax.axis_index("c")
    s = lax.axis_index("s")
    t = c * NS + s
    bufs = (b0, b1, b2, b3, b4, b5, b6, b7)

    pltpu.sync_copy(rows_hbm.at[pl.ds(t * PER_TILE, PER_TILE)], idx_v)
    _zero_vmem(hist, NPAD)

    ones16 = jnp.ones((16,), _f32)

    def scan(g, ms):
        idx16 = idx_v[pl.ds(g * 16, 16)]
        plsc.addupdate_scatter(hist, [idx16], ones16)
        basket = t * (PER_TILE // 16) + g
        packed = idx16 * 8192 + basket
        chunk = idx16 // C
        new_ms = []
        for k in range(NCHUNK):
            mask = chunk == k
            plsc.store_compressed(bufs[k].at[pl.ds(ms[k], 16)], packed,
                                  mask=mask)
            new_ms.append(ms[k] + jnp.sum(mask.astype(_i32)))
        return tuple(new_ms)

    ms = lax.fori_loop(0, PER_TILE // 16, scan, (0,) * NCHUNK)

    lane = lax.iota(_i32, 16)
    mp = []
    for k in range(NCHUNK):
        trash = jnp.full((16,), ((k + 1) * C) * 8192, _i32)
        for j in range(8):
            bufs[k][pl.ds(ms[k] + j * 16, 16)] = trash
        mp.append(((ms[k] + 127) // 128) * 128)

    cvec = jnp.full((16,), 0, _i32)
    for k in range(NCHUNK):
        cvec = jnp.where(lane == k, mp[k], cvec)
    cnt_v[...] = cvec
    pltpu.sync_copy(cnt_v, counts_hbm.at[pl.ds(t * 16, 16)])
    for k in range(NCHUNK):
        pltpu.sync_copy(bufs[k],
                        lists_hbm.at[pl.ds((t * NCHUNK + k) * LCAP, LCAP)])

    pltpu.sync_copy(hist, dvp_hbm.at[pl.ds(t * NPAD, NPAD)])


_k1 = pl.kernel(
    _k1_body,
    out_type=[
        jax.ShapeDtypeStruct((NW * NPAD,), _f32),
        jax.ShapeDtypeStruct((NW * NCHUNK * LCAP,), _i32),
        jax.ShapeDtypeStruct((NW * 16,), _i32),
    ],
    mesh=_mesh,
    compiler_params=_sc_params,
    scratch_types=[
        pltpu.VMEM((PER_TILE,), _i32),
        pltpu.VMEM((NPAD,), _f32),
        *[pltpu.VMEM((LCAP,), _i32) for _ in range(NCHUNK)],
        pltpu.VMEM((16,), _i32),
        pltpu.SemaphoreType.DMA,
    ],
)


def _k2a_body(*refs):
    dv2_ref = refs[-1]
    dv = refs[0][...]
    for r in refs[1:-1]:
        dv = dv + r[...]
    dv2_ref[...] = jnp.where(dv > 0, lax.rsqrt(jnp.maximum(dv, 1.0)), 0.0)


def _k2a(dvp_flat):
    nblk = NPAD // 1024
    return pl.pallas_call(
        _k2a_body,
        grid=(nblk,),
        in_specs=[
            pl.BlockSpec((1024,), lambda i, t=t: (i + t * nblk,))
            for t in range(NW)
        ],
        out_specs=pl.BlockSpec((1024,), lambda i: (i,)),
        out_shape=jax.ShapeDtypeStruct((NPAD,), _f32),
    )(*([dvp_flat] * NW))


def _k2b_body(x0_ref, dv2_ref, z0_ref, l2_ref):
    i = pl.program_id(0)
    x0 = x0_ref[...]
    z0_ref[...] = x0 * dv2_ref[...].reshape(128, 1)

    @pl.when(i == 0)
    def _():
        l2_ref[0, 0] = 0.0

    l2_ref[0, 0] += jnp.sum(x0 * x0)


def _k2b(x0p, dv2_flat):
    return pl.pallas_call(
        _k2b_body,
        grid=(NPAD // 128,),
        in_specs=[
            pl.BlockSpec((128, EMB), lambda i: (i, 0)),
            pl.BlockSpec((128,), lambda i: (i,)),
        ],
        out_specs=[
            pl.BlockSpec((128, EMB), lambda i: (i, 0)),
            pl.BlockSpec(memory_space=pltpu.SMEM),
        ],
        out_shape=[
            jax.ShapeDtypeStruct((NPAD, EMB), _f32),
            jax.ShapeDtypeStruct((1, 1), _f32),
        ],
    )(x0p, dv2_flat)


def _t_body(z_hbm, rows_hbm, b_hbm, idx_v, rows_v, out_v, sem):
    c = lax.axis_index("c")
    s = lax.axis_index("s")
    t = c * NS + s
    pltpu.sync_copy(rows_hbm.at[pl.ds(t * PER_TILE, PER_TILE)], idx_v)

    def batch(j, _):
        pltpu.async_copy(z_hbm.at[idx_v.at[pl.ds(j * 128, 128)]], rows_v,
                         sem).wait()
        for bb in range(8):
            for cc in range(8):
                acc = rows_v[bb * 16, pl.ds(cc * 16, 16)]
                for k in range(1, 16):
                    acc = acc + rows_v[bb * 16 + k, pl.ds(cc * 16, 16)]
                out_v[j * 8 + bb, pl.ds(cc * 16, 16)] = acc * 0.25
        return 0

    lax.fori_loop(0, PER_TILE // 128, batch, 0)
    pltpu.sync_copy(out_v, b_hbm.at[pl.ds(t * (E // NW), E // NW)])


_t_kernel = pl.kernel(
    _t_body,
    out_type=jax.ShapeDtypeStruct((E, EMB), _f32),
    mesh=_mesh,
    compiler_params=_sc_params,
    scratch_types=[
        pltpu.VMEM((PER_TILE,), _i32),
        pltpu.VMEM((128, EMB), _f32),
        pltpu.VMEM((E // NW, EMB), _f32),
        pltpu.SemaphoreType.DMA,
    ],
)


def _s_scatter_pass(lists_hbm, counts_hbm, bvals_hbm, k,
                    list_v, cnt_v, lrow2d, col2d, rows_v, zero_v, acc, sem):
    s = lax.axis_index("s")
    lo = k * C

    for q in range((ZR + 31) // 32):
        sz = min(32, ZR - q * 32)
        pltpu.sync_copy(zero_v.at[pl.ds(0, sz)],
                        acc.at[pl.ds(s * ZR + q * 32, sz)])
    plsc.subcore_barrier()

    for rr in range(2):
        r = s * 2 + rr
        pltpu.sync_copy(counts_hbm.at[pl.ds(r * 16, 16)], cnt_v)
        pltpu.sync_copy(lists_hbm.at[pl.ds((r * NCHUNK + k) * LCAP, LCAP)],
                        list_v)
        cv = cnt_v[...]
        nb = jnp.sum(jnp.where(lax.iota(_i32, 16) == k, cv, 0)) // 128

        def batch(i, _):
            for tt in range(8):
                v16 = list_v[pl.ds(i * 128 + tt * 16, 16)]
                col2d[0, pl.ds(tt * 16, 16)] = jnp.bitwise_and(v16, 8191)
                lrow2d[0, pl.ds(tt * 16, 16)] = (
                    lax.shift_right_logical(v16, 13) - lo)
            pltpu.async_copy(bvals_hbm.at[col2d.at[0]], rows_v, sem).wait()
            pltpu.sync_copy(rows_v, acc.at[lrow2d.at[0]], add=True)
            return 0

        lax.fori_loop(0, nb, batch, 0)
    plsc.subcore_barrier()


def _s_body(mode, lists_hbm, counts_hbm, bvals_hbm, dv2_hbm, x0_hbm, x1_hbm,
            o1_hbm, o2_hbm,
            list_v, cnt_v, lrow2d, col2d, rows_v, zero_v, dv2_v,
            sbuf, obuf1, obuf2, xbuf0, xbuf1, acc, sem):
    c = lax.axis_index("c")
    s = lax.axis_index("s")

    def z128(i, _):
        zero_v[i // 8, pl.ds((i % 8) * 16, 16)] = jnp.zeros((16,), _f32)
        return 0

    lax.fori_loop(0, 32 * 8, z128, 0)

    for p in range(NCHUNK // NC):
        k = p * NC + c
        _s_scatter_pass(lists_hbm, counts_hbm, bvals_hbm, k,
                        list_v, cnt_v, lrow2d, col2d, rows_v, zero_v, acc,
                        sem)
        lo = k * C
        g0 = lo + s * WB
        pltpu.sync_copy(dv2_hbm.at[pl.ds(g0, WB)], dv2_v.at[pl.ds(0, WB)])
        for q in range((WB + 63) // 64):
            sz = min(64, WB - q * 64)
            pltpu.sync_copy(acc.at[pl.ds(s * WB + q * 64, sz)],
                            sbuf.at[pl.ds(0, sz)])
            if mode == 1:
                pltpu.sync_copy(x0_hbm.at[pl.ds(g0 + q * 64, sz)],
                                xbuf0.at[pl.ds(0, sz)])
                pltpu.sync_copy(x1_hbm.at[pl.ds(g0 + q * 64, sz)],
                                xbuf1.at[pl.ds(0, sz)])

            def wrow(r, _):
                w = dv2_v[pl.ds(q * 64 + r, 16)][0]
                for cc in range(8):
                    srow = sbuf[r, pl.ds(cc * 16, 16)]
                    xv = srow * (w * 0.25)
                    if mode == 0:
                        obuf1[r, pl.ds(cc * 16, 16)] = xv
                        obuf2[r, pl.ds(cc * 16, 16)] = xv * w
                    else:
                        obuf1[r, pl.ds(cc * 16, 16)] = (
                            xbuf0[r, pl.ds(cc * 16, 16)]
                            + xbuf1[r, pl.ds(cc * 16, 16)] + xv) * (1.0 / 3.0)
                return 0

            lax.fori_loop(0, sz, wrow, 0)
            pltpu.sync_copy(obuf1.at[pl.ds(0, sz)],
                            o1_hbm.at[pl.ds(g0 + q * 64, sz)])
            if mode == 0:
                pltpu.sync_copy(obuf2.at[pl.ds(0, sz)],
                                o2_hbm.at[pl.ds(g0 + q * 64, sz)])
        plsc.subcore_barrier()


def _make_s_kernel(mode):
    body = functools.partial(_s_body, mode)
    if mode == 0:
        def body_wrap(lists_hbm, counts_hbm, bvals_hbm, dv2_hbm,
                      o1_hbm, o2_hbm,
                      list_v, cnt_v, lrow2d, col2d, rows_v, zero_v, dv2_v,
                      sbuf, obuf1, obuf2, acc, sem):
            return body(lists_hbm, counts_hbm, bvals_hbm, dv2_hbm,
                        None, None, o1_hbm, o2_hbm,
                        list_v, cnt_v, lrow2d, col2d, rows_v, zero_v, dv2_v,
                        sbuf, obuf1, obuf2, None, None, acc, sem)
        out_type = [jax.ShapeDtypeStruct((NPAD, EMB), _f32)] * 2
    else:
        def body_wrap(lists_hbm, counts_hbm, bvals_hbm, dv2_hbm,
                      x0_hbm, x1_hbm, o1_hbm,
                      list_v, cnt_v, lrow2d, col2d, rows_v, zero_v, dv2_v,
                      sbuf, obuf1, xbuf0, xbuf1, acc, sem):
            return body(lists_hbm, counts_hbm, bvals_hbm, dv2_hbm,
                        x0_hbm, x1_hbm, o1_hbm, None,
                        list_v, cnt_v, lrow2d, col2d, rows_v, zero_v, dv2_v,
                        sbuf, obuf1, None, xbuf0, xbuf1, acc, sem)
        out_type = jax.ShapeDtypeStruct((NPAD, EMB), _f32)
    return pl.kernel(
        body_wrap,
        out_type=out_type,
        mesh=_mesh,
        compiler_params=_sc_params,
        scratch_types=[
            pltpu.VMEM((LCAP,), _i32),
            pltpu.VMEM((16,), _i32),
            pltpu.VMEM((1, 128), _i32),
            pltpu.VMEM((1, 128), _i32),
            pltpu.VMEM((128, EMB), _f32),
            pltpu.VMEM((32, EMB), _f32),
            pltpu.VMEM((WB + 16,), _f32),
            pltpu.VMEM((64, EMB), _f32),
            pltpu.VMEM((64, EMB), _f32),
            pltpu.VMEM((64, EMB), _f32),
            *([pltpu.VMEM((64, EMB), _f32)] if mode == 1 else []),
            pltpu.VMEM_SHARED((NS * ZR, EMB), _f32),
            pltpu.SemaphoreType.DMA,
        ],
    )


_s1_kernel = _make_s_kernel(0)
_s2_kernel = _make_s_kernel(1)


def _g_body(ir_hbm, b1_hbm, b2_hbm, pos_hbm, neg_hbm, nbsk_hbm,
            posrep, negrep, nb1, nb2,
            idx_v, nidx_v, rows_v, sem):
    c = lax.axis_index("c")
    s = lax.axis_index("s")
    t = c * NS + s
    npt = (B * BSK) // NW

    for which in range(2):
        src = pos_hbm if which == 0 else neg_hbm
        dst = posrep if which == 0 else negrep
        pltpu.sync_copy(src.at[pl.ds(t * npt, npt)], idx_v)

        def batch(i, _):
            pltpu.async_copy(ir_hbm.at[idx_v.at[pl.ds(i * 128, 128)]],
                             rows_v, sem).wait()
            pltpu.sync_copy(rows_v,
                            dst.at[pl.ds(t * npt + i * 128, 128)])
            return 0

        lax.fori_loop(0, npt // 128, batch, 0)

    nbp = B // NW
    pltpu.sync_copy(nbsk_hbm.at[pl.ds(t * nbp, nbp)], nidx_v)
    for which in range(2):
        src = b1_hbm if which == 0 else b2_hbm
        dst = nb1 if which == 0 else nb2
        pltpu.async_copy(src.at[nidx_v], rows_v.at[pl.ds(0, nbp)],
                         sem).wait()
        pltpu.sync_copy(rows_v.at[pl.ds(0, nbp)],
                        dst.at[pl.ds(t * nbp, nbp)])


_g_kernel = pl.kernel(
    _g_body,
    out_type=[
        jax.ShapeDtypeStruct((B * BSK, EMB), _f32),
        jax.ShapeDtypeStruct((B * BSK, EMB), _f32),
        jax.ShapeDtypeStruct((B, EMB), _f32),
        jax.ShapeDtypeStruct((B, EMB), _f32),
    ],
    mesh=_mesh,
    compiler_params=_sc_params,
    scratch_types=[
        pltpu.VMEM(((B * BSK) // NW,), _i32),
        pltpu.VMEM((B // NW,), _i32),
        pltpu.VMEM((128, EMB), _f32),
        pltpu.SemaphoreType.DMA,
    ],
)


def _k8_body(b1_ref, b2_ref, pos_ref, neg_ref, nb1_ref, nb2_ref, l2_ref,
             out_ref, accs):
    i = pl.program_id(0)
    ub = 128

    @pl.when(i == 0)
    def _():
        accs[0] = 0.0
        accs[1] = 0.0

    br = (b1_ref[...] + b2_ref[...]) * 0.5
    br3 = br.reshape(ub, NB, EMB)
    user7 = jnp.sum(br3[:, : NB - 1, :], axis=1) * (1.0 / (NB - 1))
    posb = br3[:, NB - 1, :]
    negb = (nb1_ref[...] + nb2_ref[...]) * 0.5

    eps = 1e-08
    y_ui = jnp.sum(user7 * posb, axis=-1)
    y_uj = jnp.sum(user7 * negb, axis=-1)
    r = jax.nn.sigmoid(y_ui - y_uj)
    r = jnp.where(r == 0.0, eps, r)
    accs[0] += jnp.sum(jnp.log(r))

    pos3 = pos_ref[...].reshape(ub, BSK, EMB)
    neg3 = neg_ref[...].reshape(ub, BSK, EMB)
    yui = jnp.sum(user7[:, None, :] * pos3, axis=-1)
    yuj = jnp.sum(user7[:, None, :] * neg3, axis=-1)
    r2 = jax.nn.sigmoid(yui - yuj)
    r2 = jnp.where(r2 == 0.0, eps, r2)
    accs[1] += jnp.sum(jnp.mean(jnp.log(r2), axis=1))

    lane = lax.broadcasted_iota(_i32, (1, 128), 1)
    loss1 = -accs[0] / B
    loss2 = -accs[1] / B
    l2reg = 0.0001 * (l2_ref[0, 0] / 2.0)
    out_ref[...] = jnp.where(lane == 0, loss1,
                             jnp.where(lane == 1, loss2, l2reg))


def _k8(b1, b2, posrep, negrep, nb1, nb2, l2):
    ub = 128
    return pl.pallas_call(
        _k8_body,
        grid=(B // ub,),
        in_specs=[
            pl.BlockSpec((ub * NB, EMB), lambda i: (i, 0)),
            pl.BlockSpec((ub * NB, EMB), lambda i: (i, 0)),
            pl.BlockSpec((ub * BSK, EMB), lambda i: (i, 0)),
            pl.BlockSpec((ub * BSK, EMB), lambda i: (i, 0)),
            pl.BlockSpec((ub, EMB), lambda i: (i, 0)),
            pl.BlockSpec((ub, EMB), lambda i: (i, 0)),
            pl.BlockSpec(memory_space=pltpu.SMEM),
        ],
        out_specs=pl.BlockSpec((1, 128), lambda i: (0, 0)),
        out_shape=jax.ShapeDtypeStruct((1, 128), _f32),
        scratch_shapes=[pltpu.SMEM((2,), _f32)],
    )(b1, b2, posrep, negrep, nb1, nb2, l2)


def kernel(log_seqs, batch_user_list, item_emb, neg_bsk_idx, neg_items):
    rows = log_seqs.reshape(-1).astype(_i32)
    x0p = jnp.concatenate(
        [item_emb, jnp.zeros((NPAD - N, EMB), _f32)], axis=0)

    dvp, lists, counts = _k1(rows)
    dv2_flat = _k2a(dvp)
    z0, l2 = _k2b(x0p, dv2_flat)

    b1 = _t_kernel(z0, rows)
    x1, z1 = _s1_kernel(lists, counts, b1, dv2_flat)
    b2 = _t_kernel(z1, rows)
    ir = _s2_kernel(lists, counts, b2, dv2_flat, x0p, x1)

    pos_idx = log_seqs[:, NB - 1].reshape(-1).astype(_i32)
    neg_idx = neg_items.reshape(-1).astype(_i32)
    posrep, negrep, nb1, nb2 = _g_kernel(ir, b1, b2, pos_idx, neg_idx,
                                         neg_bsk_idx.astype(_i32))

    out = _k8(b1, b2, posrep, negrep, nb1, nb2, l2)
    return (out[0, 0], out[0, 1], out[0, 2])


# speedup vs baseline: 3.0303x; 1.1011x over previous
import functools

import jax
import jax.numpy as jnp
from jax import lax
from jax.experimental import pallas as pl
from jax.experimental.pallas import tpu as pltpu
from jax.experimental.pallas import tpu_sc as plsc

NUM_ITEM = 50000
EMB = 128
PAD = 50000
B = 1024
NB = 8
BSK = 16
E = B * NB
NE = E * BSK
N = NUM_ITEM + 1
NPAD = 50176
NCHUNK = 8
C = NPAD // NCHUNK
CACC = C + 128
LCAP = 4096 + 128
NC = 2
NS = 16
NW = NC * NS
PER_TILE = NE // NW
HSLC = NPAD // NS
WB = C // NS
ZR = CACC // NS

_mesh = plsc.VectorSubcoreMesh(core_axis_name="c", subcore_axis_name="s")
_sc_params = pltpu.CompilerParams(needs_layout_passes=False)
_f32 = jnp.float32
_i32 = jnp.int32


def _zero_vmem(ref, n):
    z = jnp.zeros((16,), ref.dtype)

    def body(i, _):
        ref[pl.ds(i * 16, 16)] = z
        return 0

    lax.fori_loop(0, n // 16, body, 0)


def _k1_body(rows_hbm, dvp_hbm, lists_hbm, counts_hbm,
             idx_v, hist, b0, b1, b2, b3, b4, b5, b6, b7, cnt_v, sem):
    c = lax.axis_index("c")
    s = lax.axis_index("s")
    t = c * NS + s
    bufs = (b0, b1, b2, b3, b4, b5, b6, b7)

    pltpu.sync_copy(rows_hbm.at[pl.ds(t * PER_TILE, PER_TILE)], idx_v)
    _zero_vmem(hist, NPAD)

    ones16 = jnp.ones((16,), _f32)

    def scan(g, ms):
        idx16 = idx_v[pl.ds(g * 16, 16)]
        plsc.addupdate_scatter(hist, [idx16], ones16)
        basket = t * (PER_TILE // 16) + g
        packed = idx16 * 8192 + basket
        chunk = idx16 // C
        new_ms = []
        for k in range(NCHUNK):
            mask = chunk == k
            plsc.store_compressed(bufs[k].at[pl.ds(ms[k], 16)], packed,
                                  mask=mask)
            new_ms.append(ms[k] + jnp.sum(mask.astype(_i32)))
        return tuple(new_ms)

    ms = lax.fori_loop(0, PER_TILE // 16, scan, (0,) * NCHUNK)

    lane = lax.iota(_i32, 16)
    mp = []
    for k in range(NCHUNK):
        trash = jnp.full((16,), ((k + 1) * C) * 8192, _i32)
        for j in range(8):
            bufs[k][pl.ds(ms[k] + j * 16, 16)] = trash
        mp.append(((ms[k] + 127) // 128) * 128)

    cvec = jnp.full((16,), 0, _i32)
    for k in range(NCHUNK):
        cvec = jnp.where(lane == k, mp[k], cvec)
    cnt_v[...] = cvec
    pltpu.sync_copy(cnt_v, counts_hbm.at[pl.ds(t * 16, 16)])
    for k in range(NCHUNK):
        pltpu.sync_copy(bufs[k],
                        lists_hbm.at[pl.ds((t * NCHUNK + k) * LCAP, LCAP)])

    pltpu.sync_copy(hist, dvp_hbm.at[pl.ds(t * NPAD, NPAD)])


_k1 = pl.kernel(
    _k1_body,
    out_type=[
        jax.ShapeDtypeStruct((NW * NPAD,), _f32),
        jax.ShapeDtypeStruct((NW * NCHUNK * LCAP,), _i32),
        jax.ShapeDtypeStruct((NW * 16,), _i32),
    ],
    mesh=_mesh,
    compiler_params=_sc_params,
    scratch_types=[
        pltpu.VMEM((PER_TILE,), _i32),
        pltpu.VMEM((NPAD,), _f32),
        *[pltpu.VMEM((LCAP,), _i32) for _ in range(NCHUNK)],
        pltpu.VMEM((16,), _i32),
        pltpu.SemaphoreType.DMA,
    ],
)


def _k2a_body(*refs):
    dv2_ref = refs[-1]
    dv = refs[0][...]
    for r in refs[1:-1]:
        dv = dv + r[...]
    dv2_ref[...] = jnp.where(dv > 0, lax.rsqrt(jnp.maximum(dv, 1.0)), 0.0)


def _k2a(dvp_flat):
    nblk = NPAD // 1024
    return pl.pallas_call(
        _k2a_body,
        grid=(nblk,),
        in_specs=[
            pl.BlockSpec((1024,), lambda i, t=t: (i + t * nblk,))
            for t in range(NW)
        ],
        out_specs=pl.BlockSpec((1024,), lambda i: (i,)),
        out_shape=jax.ShapeDtypeStruct((NPAD,), _f32),
    )(*([dvp_flat] * NW))


def _k2b_body(x0_ref, dv2_ref, z0_ref, l2_ref):
    i = pl.program_id(0)
    x0 = x0_ref[...]
    z0_ref[...] = x0 * dv2_ref[...].reshape(128, 1)

    @pl.when(i == 0)
    def _():
        l2_ref[0, 0] = 0.0

    l2_ref[0, 0] += jnp.sum(x0 * x0)


def _k2b(x0p, dv2_flat):
    return pl.pallas_call(
        _k2b_body,
        grid=(NPAD // 128,),
        in_specs=[
            pl.BlockSpec((128, EMB), lambda i: (i, 0)),
            pl.BlockSpec((128,), lambda i: (i,)),
        ],
        out_specs=[
            pl.BlockSpec((128, EMB), lambda i: (i, 0)),
            pl.BlockSpec(memory_space=pltpu.SMEM),
        ],
        out_shape=[
            jax.ShapeDtypeStruct((NPAD, EMB), _f32),
            jax.ShapeDtypeStruct((1, 1), _f32),
        ],
    )(x0p, dv2_flat)


def _t_body(z_hbm, rows_hbm, b_hbm, idx_v, r0, r1, bidx, zero_v, acc_sh,
            sem0, sem1):
    c = lax.axis_index("c")
    s = lax.axis_index("s")
    t = c * NS + s
    nbk = E // NW
    pltpu.sync_copy(rows_hbm.at[pl.ds(t * PER_TILE, PER_TILE)], idx_v)

    def zv(i, _):
        zero_v[i // 8, pl.ds((i % 8) * 16, 16)] = jnp.zeros((16,), _f32)
        return 0

    lax.fori_loop(0, 32 * 8, zv, 0)
    for q in range(nbk // 32):
        pltpu.sync_copy(zero_v, acc_sh.at[pl.ds(s * nbk + q * 32, 32)])

    nb = PER_TILE // 128
    bufs = (r0, r1)
    sems = (sem0, sem1)
    cps = {0: pltpu.async_copy(z_hbm.at[idx_v.at[pl.ds(0, 128)]], r0, sem0)}
    for j in range(nb):
        if j + 1 < nb:
            cps[j + 1] = pltpu.async_copy(
                z_hbm.at[idx_v.at[pl.ds((j + 1) * 128, 128)]],
                bufs[(j + 1) % 2], sems[(j + 1) % 2])
        cps[j].wait()
        base = s * nbk + j * 8
        for q in range(8):
            bidx[0, pl.ds(q * 16, 16)] = jnp.full((16,), 0, _i32) + (base + q)
        pltpu.sync_copy(bufs[j % 2], acc_sh.at[bidx.at[0]], add=True)

    pltpu.sync_copy(acc_sh.at[pl.ds(s * nbk, nbk)],
                    b_hbm.at[pl.ds(t * nbk, nbk)])


_t_kernel = pl.kernel(
    _t_body,
    out_type=jax.ShapeDtypeStruct((E, EMB), _f32),
    mesh=_mesh,
    compiler_params=_sc_params,
    scratch_types=[
        pltpu.VMEM((PER_TILE,), _i32),
        pltpu.VMEM((128, EMB), _f32),
        pltpu.VMEM((128, EMB), _f32),
        pltpu.VMEM((1, 128), _i32),
        pltpu.VMEM((32, EMB), _f32),
        pltpu.VMEM_SHARED((NS * (E // NW), EMB), _f32),
        pltpu.SemaphoreType.DMA,
        pltpu.SemaphoreType.DMA,
    ],
)


def _s_scatter_pass(lists_hbm, counts_hbm, bvals_hbm, k,
                    list_v, cnt_v, lrow2d, col2d, rows_v, zero_v, acc, sem):
    s = lax.axis_index("s")
    lo = k * C

    for q in range((ZR + 31) // 32):
        sz = min(32, ZR - q * 32)
        pltpu.sync_copy(zero_v.at[pl.ds(0, sz)],
                        acc.at[pl.ds(s * ZR + q * 32, sz)])
    plsc.subcore_barrier()

    for rr in range(2):
        r = s * 2 + rr
        pltpu.sync_copy(counts_hbm.at[pl.ds(r * 16, 16)], cnt_v)
        pltpu.sync_copy(lists_hbm.at[pl.ds((r * NCHUNK + k) * LCAP, LCAP)],
                        list_v)
        cv = cnt_v[...]
        nb = jnp.sum(jnp.where(lax.iota(_i32, 16) == k, cv, 0)) // 128

        def batch(i, _):
            for tt in range(8):
                v16 = list_v[pl.ds(i * 128 + tt * 16, 16)]
                col2d[0, pl.ds(tt * 16, 16)] = jnp.bitwise_and(v16, 8191)
                lrow2d[0, pl.ds(tt * 16, 16)] = (
                    lax.shift_right_logical(v16, 13) - lo)
            pltpu.async_copy(bvals_hbm.at[col2d.at[0]], rows_v, sem).wait()
            pltpu.sync_copy(rows_v, acc.at[lrow2d.at[0]], add=True)
            return 0

        lax.fori_loop(0, nb, batch, 0)
    plsc.subcore_barrier()


def _s_body(mode, lists_hbm, counts_hbm, bvals_hbm, dv2_hbm, x0_hbm, x1_hbm,
            o1_hbm, o2_hbm,
            list_v, cnt_v, lrow2d, col2d, rows_v, zero_v, dv2_v,
            sbuf, obuf1, obuf2, xbuf0, xbuf1, acc, sem):
    c = lax.axis_index("c")
    s = lax.axis_index("s")

    def z128(i, _):
        zero_v[i // 8, pl.ds((i % 8) * 16, 16)] = jnp.zeros((16,), _f32)
        return 0

    lax.fori_loop(0, 32 * 8, z128, 0)

    for p in range(NCHUNK // NC):
        k = p * NC + c
        _s_scatter_pass(lists_hbm, counts_hbm, bvals_hbm, k,
                        list_v, cnt_v, lrow2d, col2d, rows_v, zero_v, acc,
                        sem)
        lo = k * C
        g0 = lo + s * WB
        pltpu.sync_copy(dv2_hbm.at[pl.ds(g0, WB)], dv2_v.at[pl.ds(0, WB)])
        for q in range((WB + 63) // 64):
            sz = min(64, WB - q * 64)
            pltpu.sync_copy(acc.at[pl.ds(s * WB + q * 64, sz)],
                            sbuf.at[pl.ds(0, sz)])
            if mode == 1:
                pltpu.sync_copy(x0_hbm.at[pl.ds(g0 + q * 64, sz)],
                                xbuf0.at[pl.ds(0, sz)])
                pltpu.sync_copy(x1_hbm.at[pl.ds(g0 + q * 64, sz)],
                                xbuf1.at[pl.ds(0, sz)])

            def wrow(r, _):
                w = dv2_v[pl.ds(q * 64 + r, 16)][0]
                for cc in range(8):
                    srow = sbuf[r, pl.ds(cc * 16, 16)]
                    xv = srow * (w * 0.0625)
                    if mode == 0:
                        obuf1[r, pl.ds(cc * 16, 16)] = xv
                        obuf2[r, pl.ds(cc * 16, 16)] = xv * w
                    else:
                        obuf1[r, pl.ds(cc * 16, 16)] = (
                            xbuf0[r, pl.ds(cc * 16, 16)]
                            + xbuf1[r, pl.ds(cc * 16, 16)] + xv) * (1.0 / 3.0)
                return 0

            lax.fori_loop(0, sz, wrow, 0)
            pltpu.sync_copy(obuf1.at[pl.ds(0, sz)],
                            o1_hbm.at[pl.ds(g0 + q * 64, sz)])
            if mode == 0:
                pltpu.sync_copy(obuf2.at[pl.ds(0, sz)],
                                o2_hbm.at[pl.ds(g0 + q * 64, sz)])
        plsc.subcore_barrier()


def _make_s_kernel(mode):
    body = functools.partial(_s_body, mode)
    if mode == 0:
        def body_wrap(lists_hbm, counts_hbm, bvals_hbm, dv2_hbm,
                      o1_hbm, o2_hbm,
                      list_v, cnt_v, lrow2d, col2d, rows_v, zero_v, dv2_v,
                      sbuf, obuf1, obuf2, acc, sem):
            return body(lists_hbm, counts_hbm, bvals_hbm, dv2_hbm,
                        None, None, o1_hbm, o2_hbm,
                        list_v, cnt_v, lrow2d, col2d, rows_v, zero_v, dv2_v,
                        sbuf, obuf1, obuf2, None, None, acc, sem)
        out_type = [jax.ShapeDtypeStruct((NPAD, EMB), _f32)] * 2
    else:
        def body_wrap(lists_hbm, counts_hbm, bvals_hbm, dv2_hbm,
                      x0_hbm, x1_hbm, o1_hbm,
                      list_v, cnt_v, lrow2d, col2d, rows_v, zero_v, dv2_v,
                      sbuf, obuf1, xbuf0, xbuf1, acc, sem):
            return body(lists_hbm, counts_hbm, bvals_hbm, dv2_hbm,
                        x0_hbm, x1_hbm, o1_hbm, None,
                        list_v, cnt_v, lrow2d, col2d, rows_v, zero_v, dv2_v,
                        sbuf, obuf1, None, xbuf0, xbuf1, acc, sem)
        out_type = jax.ShapeDtypeStruct((NPAD, EMB), _f32)
    return pl.kernel(
        body_wrap,
        out_type=out_type,
        mesh=_mesh,
        compiler_params=_sc_params,
        scratch_types=[
            pltpu.VMEM((LCAP,), _i32),
            pltpu.VMEM((16,), _i32),
            pltpu.VMEM((1, 128), _i32),
            pltpu.VMEM((1, 128), _i32),
            pltpu.VMEM((128, EMB), _f32),
            pltpu.VMEM((32, EMB), _f32),
            pltpu.VMEM((WB + 16,), _f32),
            pltpu.VMEM((64, EMB), _f32),
            pltpu.VMEM((64, EMB), _f32),
            pltpu.VMEM((64, EMB), _f32),
            *([pltpu.VMEM((64, EMB), _f32)] if mode == 1 else []),
            pltpu.VMEM_SHARED((NS * ZR, EMB), _f32),
            pltpu.SemaphoreType.DMA,
        ],
    )


_s1_kernel = _make_s_kernel(0)
_s2_kernel = _make_s_kernel(1)


def _g_body(ir_hbm, b1_hbm, b2_hbm, pos_hbm, neg_hbm, nbsk_hbm,
            posrep, negrep, nb1, nb2,
            idx_v, nidx_v, rows_v, sem):
    c = lax.axis_index("c")
    s = lax.axis_index("s")
    t = c * NS + s
    npt = (B * BSK) // NW

    for which in range(2):
        src = pos_hbm if which == 0 else neg_hbm
        dst = posrep if which == 0 else negrep
        pltpu.sync_copy(src.at[pl.ds(t * npt, npt)], idx_v)

        def batch(i, _):
            pltpu.async_copy(ir_hbm.at[idx_v.at[pl.ds(i * 128, 128)]],
                             rows_v, sem).wait()
            pltpu.sync_copy(rows_v,
                            dst.at[pl.ds(t * npt + i * 128, 128)])
            return 0

        lax.fori_loop(0, npt // 128, batch, 0)

    nbp = B // NW
    pltpu.sync_copy(nbsk_hbm.at[pl.ds(t * nbp, nbp)], nidx_v)
    for which in range(2):
        src = b1_hbm if which == 0 else b2_hbm
        dst = nb1 if which == 0 else nb2
        pltpu.async_copy(src.at[nidx_v], rows_v.at[pl.ds(0, nbp)],
                         sem).wait()
        pltpu.sync_copy(rows_v.at[pl.ds(0, nbp)],
                        dst.at[pl.ds(t * nbp, nbp)])


_g_kernel = pl.kernel(
    _g_body,
    out_type=[
        jax.ShapeDtypeStruct((B * BSK, EMB), _f32),
        jax.ShapeDtypeStruct((B * BSK, EMB), _f32),
        jax.ShapeDtypeStruct((B, EMB), _f32),
        jax.ShapeDtypeStruct((B, EMB), _f32),
    ],
    mesh=_mesh,
    compiler_params=_sc_params,
    scratch_types=[
        pltpu.VMEM(((B * BSK) // NW,), _i32),
        pltpu.VMEM((B // NW,), _i32),
        pltpu.VMEM((128, EMB), _f32),
        pltpu.SemaphoreType.DMA,
    ],
)


def _k8_body(b1_ref, b2_ref, pos_ref, neg_ref, nb1_ref, nb2_ref, l2_ref,
             out_ref, accs):
    i = pl.program_id(0)
    ub = 128

    @pl.when(i == 0)
    def _():
        accs[0] = 0.0
        accs[1] = 0.0

    br = (b1_ref[...] + b2_ref[...]) * 0.125
    br3 = br.reshape(ub, NB, EMB)
    user7 = jnp.sum(br3[:, : NB - 1, :], axis=1) * (1.0 / (NB - 1))
    posb = br3[:, NB - 1, :]
    negb = (nb1_ref[...] + nb2_ref[...]) * 0.125

    eps = 1e-08
    y_ui = jnp.sum(user7 * posb, axis=-1)
    y_uj = jnp.sum(user7 * negb, axis=-1)
    r = jax.nn.sigmoid(y_ui - y_uj)
    r = jnp.where(r == 0.0, eps, r)
    accs[0] += jnp.sum(jnp.log(r))

    pos3 = pos_ref[...].reshape(ub, BSK, EMB)
    neg3 = neg_ref[...].reshape(ub, BSK, EMB)
    yui = jnp.sum(user7[:, None, :] * pos3, axis=-1)
    yuj = jnp.sum(user7[:, None, :] * neg3, axis=-1)
    r2 = jax.nn.sigmoid(yui - yuj)
    r2 = jnp.where(r2 == 0.0, eps, r2)
    accs[1] += jnp.sum(jnp.mean(jnp.log(r2), axis=1))

    lane = lax.broadcasted_iota(_i32, (1, 128), 1)
    loss1 = -accs[0] / B
    loss2 = -accs[1] / B
    l2reg = 0.0001 * (l2_ref[0, 0] / 2.0)
    out_ref[...] = jnp.where(lane == 0, loss1,
                             jnp.where(lane == 1, loss2, l2reg))


def _k8(b1, b2, posrep, negrep, nb1, nb2, l2):
    ub = 128
    return pl.pallas_call(
        _k8_body,
        grid=(B // ub,),
        in_specs=[
            pl.BlockSpec((ub * NB, EMB), lambda i: (i, 0)),
            pl.BlockSpec((ub * NB, EMB), lambda i: (i, 0)),
            pl.BlockSpec((ub * BSK, EMB), lambda i: (i, 0)),
            pl.BlockSpec((ub * BSK, EMB), lambda i: (i, 0)),
            pl.BlockSpec((ub, EMB), lambda i: (i, 0)),
            pl.BlockSpec((ub, EMB), lambda i: (i, 0)),
            pl.BlockSpec(memory_space=pltpu.SMEM),
        ],
        out_specs=pl.BlockSpec((1, 128), lambda i: (0, 0)),
        out_shape=jax.ShapeDtypeStruct((1, 128), _f32),
        scratch_shapes=[pltpu.SMEM((2,), _f32)],
    )(b1, b2, posrep, negrep, nb1, nb2, l2)


def kernel(log_seqs, batch_user_list, item_emb, neg_bsk_idx, neg_items):
    rows = log_seqs.reshape(-1).astype(_i32)
    x0p = jnp.concatenate(
        [item_emb, jnp.zeros((NPAD - N, EMB), _f32)], axis=0)

    dvp, lists, counts = _k1(rows)
    dv2_flat = _k2a(dvp)
    z0, l2 = _k2b(x0p, dv2_flat)

    b1 = _t_kernel(z0, rows)
    x1, z1 = _s1_kernel(lists, counts, b1, dv2_flat)
    b2 = _t_kernel(z1, rows)
    ir = _s2_kernel(lists, counts, b2, dv2_flat, x0p, x1)

    pos_idx = log_seqs[:, NB - 1].reshape(-1).astype(_i32)
    neg_idx = neg_items.reshape(-1).astype(_i32)
    posrep, negrep, nb1, nb2 = _g_kernel(ir, b1, b2, pos_idx, neg_idx,
                                         neg_bsk_idx.astype(_i32))

    out = _k8(b1, b2, posrep, negrep, nb1, nb2, l2)
    return (out[0, 0], out[0, 1], out[0, 2])


# speedup vs baseline: 3.0536x; 1.0077x over previous
import functools

import jax
import jax.numpy as jnp
from jax import lax
from jax.experimental import pallas as pl
from jax.experimental.pallas import tpu as pltpu
from jax.experimental.pallas import tpu_sc as plsc

NUM_ITEM = 50000
EMB = 128
PAD = 50000
B = 1024
NB = 8
BSK = 16
E = B * NB
NE = E * BSK
N = NUM_ITEM + 1
NPAD = 50176
NCHUNK = 8
C = NPAD // NCHUNK
CACC = C + 128
LCAP = 4096 + 128
NC = 2
NS = 16
NW = NC * NS
PER_TILE = NE // NW
HSLC = NPAD // NS
WB = C // NS
ZR = CACC // NS

_mesh = plsc.VectorSubcoreMesh(core_axis_name="c", subcore_axis_name="s")
_sc_params = pltpu.CompilerParams(needs_layout_passes=False)
_f32 = jnp.float32
_i32 = jnp.int32


def _zero_vmem(ref, n):
    z = jnp.zeros((16,), ref.dtype)

    def body(i, _):
        ref[pl.ds(i * 16, 16)] = z
        return 0

    lax.fori_loop(0, n // 16, body, 0)


def _k1_body(rows_hbm, dvp_hbm, lists_hbm, counts_hbm,
             idx_v, hist, b0, b1, b2, b3, b4, b5, b6, b7, cnt_v, sem):
    c = lax.axis_index("c")
    s = lax.axis_index("s")
    t = c * NS + s
    bufs = (b0, b1, b2, b3, b4, b5, b6, b7)

    pltpu.sync_copy(rows_hbm.at[pl.ds(t * PER_TILE, PER_TILE)], idx_v)
    _zero_vmem(hist, NPAD)

    ones16 = jnp.ones((16,), _f32)

    def scan(g, ms):
        idx16 = idx_v[pl.ds(g * 16, 16)]
        plsc.addupdate_scatter(hist, [idx16], ones16)
        basket = t * (PER_TILE // 16) + g
        packed = idx16 * 8192 + basket
        chunk = idx16 // C
        new_ms = []
        for k in range(NCHUNK):
            mask = chunk == k
            plsc.store_compressed(bufs[k].at[pl.ds(ms[k], 16)], packed,
                                  mask=mask)
            new_ms.append(ms[k] + jnp.sum(mask.astype(_i32)))
        return tuple(new_ms)

    ms = lax.fori_loop(0, PER_TILE // 16, scan, (0,) * NCHUNK)

    lane = lax.iota(_i32, 16)
    mp = []
    for k in range(NCHUNK):
        trash = jnp.full((16,), ((k + 1) * C) * 8192, _i32)
        for j in range(8):
            bufs[k][pl.ds(ms[k] + j * 16, 16)] = trash
        mp.append(((ms[k] + 127) // 128) * 128)

    cvec = jnp.full((16,), 0, _i32)
    for k in range(NCHUNK):
        cvec = jnp.where(lane == k, mp[k], cvec)
    cnt_v[...] = cvec
    pltpu.sync_copy(cnt_v, counts_hbm.at[pl.ds(t * 16, 16)])
    for k in range(NCHUNK):
        pltpu.sync_copy(bufs[k],
                        lists_hbm.at[pl.ds((t * NCHUNK + k) * LCAP, LCAP)])

    pltpu.sync_copy(hist, dvp_hbm.at[pl.ds(t * NPAD, NPAD)])


_k1 = pl.kernel(
    _k1_body,
    out_type=[
        jax.ShapeDtypeStruct((NW * NPAD,), _f32),
        jax.ShapeDtypeStruct((NW * NCHUNK * LCAP,), _i32),
        jax.ShapeDtypeStruct((NW * 16,), _i32),
    ],
    mesh=_mesh,
    compiler_params=_sc_params,
    scratch_types=[
        pltpu.VMEM((PER_TILE,), _i32),
        pltpu.VMEM((NPAD,), _f32),
        *[pltpu.VMEM((LCAP,), _i32) for _ in range(NCHUNK)],
        pltpu.VMEM((16,), _i32),
        pltpu.SemaphoreType.DMA,
    ],
)


def _k2a_body(*refs):
    dv2_ref = refs[-1]
    dv = refs[0][...]
    for r in refs[1:-1]:
        dv = dv + r[...]
    dv2_ref[...] = jnp.where(dv > 0, lax.rsqrt(jnp.maximum(dv, 1.0)), 0.0)


def _k2a(dvp_flat):
    nblk = NPAD // 1024
    return pl.pallas_call(
        _k2a_body,
        grid=(nblk,),
        in_specs=[
            pl.BlockSpec((1024,), lambda i, t=t: (i + t * nblk,))
            for t in range(NW)
        ],
        out_specs=pl.BlockSpec((1024,), lambda i: (i,)),
        out_shape=jax.ShapeDtypeStruct((NPAD,), _f32),
    )(*([dvp_flat] * NW))


def _k2b_body(x0_ref, dv2_ref, z0_ref, l2_ref):
    i = pl.program_id(0)
    x0 = x0_ref[...]
    z0_ref[...] = x0 * dv2_ref[...].reshape(128, 1)

    @pl.when(i == 0)
    def _():
        l2_ref[0, 0] = 0.0

    l2_ref[0, 0] += jnp.sum(x0 * x0)


def _k2b(x0p, dv2_flat):
    return pl.pallas_call(
        _k2b_body,
        grid=(NPAD // 128,),
        in_specs=[
            pl.BlockSpec((128, EMB), lambda i: (i, 0)),
            pl.BlockSpec((128,), lambda i: (i,)),
        ],
        out_specs=[
            pl.BlockSpec((128, EMB), lambda i: (i, 0)),
            pl.BlockSpec(memory_space=pltpu.SMEM),
        ],
        out_shape=[
            jax.ShapeDtypeStruct((NPAD, EMB), _f32),
            jax.ShapeDtypeStruct((1, 1), _f32),
        ],
    )(x0p, dv2_flat)


def _t_body(z_hbm, rows_hbm, b_hbm, idx_v, r0, r1, bidx, zero_v, acc_sh,
            sem0, sem1):
    c = lax.axis_index("c")
    s = lax.axis_index("s")
    t = c * NS + s
    nbk = E // NW
    pltpu.sync_copy(rows_hbm.at[pl.ds(t * PER_TILE, PER_TILE)], idx_v)

    def zv(i, _):
        zero_v[i // 8, pl.ds((i % 8) * 16, 16)] = jnp.zeros((16,), _f32)
        return 0

    lax.fori_loop(0, 32 * 8, zv, 0)
    for q in range(nbk // 32):
        pltpu.sync_copy(zero_v, acc_sh.at[pl.ds(s * nbk + q * 32, 32)])

    nb = PER_TILE // 128
    bufs = (r0, r1)
    sems = (sem0, sem1)
    cps = {0: pltpu.async_copy(z_hbm.at[idx_v.at[pl.ds(0, 128)]], r0, sem0)}
    for j in range(nb):
        if j + 1 < nb:
            cps[j + 1] = pltpu.async_copy(
                z_hbm.at[idx_v.at[pl.ds((j + 1) * 128, 128)]],
                bufs[(j + 1) % 2], sems[(j + 1) % 2])
        cps[j].wait()
        base = s * nbk + j * 8
        for q in range(8):
            bidx[0, pl.ds(q * 16, 16)] = jnp.full((16,), 0, _i32) + (base + q)
        pltpu.sync_copy(bufs[j % 2], acc_sh.at[bidx.at[0]], add=True)

    pltpu.sync_copy(acc_sh.at[pl.ds(s * nbk, nbk)],
                    b_hbm.at[pl.ds(t * nbk, nbk)])


_t_kernel = pl.kernel(
    _t_body,
    out_type=jax.ShapeDtypeStruct((E, EMB), _f32),
    mesh=_mesh,
    compiler_params=_sc_params,
    scratch_types=[
        pltpu.VMEM((PER_TILE,), _i32),
        pltpu.VMEM((128, EMB), _f32),
        pltpu.VMEM((128, EMB), _f32),
        pltpu.VMEM((1, 128), _i32),
        pltpu.VMEM((32, EMB), _f32),
        pltpu.VMEM_SHARED((NS * (E // NW), EMB), _f32),
        pltpu.SemaphoreType.DMA,
        pltpu.SemaphoreType.DMA,
    ],
)


def _s_scatter_pass(lists_hbm, counts_hbm, bvals_hbm, k,
                    list_v, cnt_v, lr0, lr1, co0, co1, r0, r1,
                    zero_v, acc, sem0, sem1):
    s = lax.axis_index("s")
    lo = k * C

    for q in range((ZR + 31) // 32):
        sz = min(32, ZR - q * 32)
        pltpu.sync_copy(zero_v.at[pl.ds(0, sz)],
                        acc.at[pl.ds(s * ZR + q * 32, sz)])
    plsc.subcore_barrier()

    lrs = (lr0, lr1)
    cos = (co0, co1)
    rbs = (r0, r1)
    sems = (sem0, sem1)

    def prep_fire(b, slot):
        for tt in range(8):
            v16 = list_v[pl.ds(b * 128 + tt * 16, 16)]
            cos[slot][0, pl.ds(tt * 16, 16)] = jnp.bitwise_and(v16, 8191)
            lrs[slot][0, pl.ds(tt * 16, 16)] = (
                lax.shift_right_logical(v16, 13) - lo)
        pltpu.async_copy(bvals_hbm.at[cos[slot].at[0]], rbs[slot],
                         sems[slot])

    for rr in range(2):
        r = s * 2 + rr
        pltpu.sync_copy(counts_hbm.at[pl.ds(r * 16, 16)], cnt_v)
        pltpu.sync_copy(lists_hbm.at[pl.ds((r * NCHUNK + k) * LCAP, LCAP)],
                        list_v)
        cv = cnt_v[...]
        nb = jnp.sum(jnp.where(lax.iota(_i32, 16) == k, cv, 0)) // 128

        @pl.when(nb > 0)
        def _():
            prep_fire(0, 0)

        @pl.when(nb > 1)
        def _():
            prep_fire(1, 1)

        def super_(i, _):
            for slot in range(2):
                b = i * 2 + slot

                @pl.when(b < nb)
                def _():
                    pltpu.make_async_copy(bvals_hbm.at[cos[slot].at[0]],
                                          rbs[slot], sems[slot]).wait()
                    pltpu.sync_copy(rbs[slot], acc.at[lrs[slot].at[0]],
                                    add=True)

                    @pl.when(b + 2 < nb)
                    def _():
                        prep_fire(b + 2, slot)
            return 0

        lax.fori_loop(0, (nb + 1) // 2, super_, 0)
    plsc.subcore_barrier()


def _s_body(mode, lists_hbm, counts_hbm, bvals_hbm, dv2_hbm, x0_hbm, x1_hbm,
            o1_hbm, o2_hbm,
            list_v, cnt_v, lr0, lr1, co0, co1, r0, r1, zero_v, dv2_v,
            sbuf, obuf1, obuf2, xbuf0, xbuf1, acc, sem0, sem1):
    c = lax.axis_index("c")
    s = lax.axis_index("s")

    def z128(i, _):
        zero_v[i // 8, pl.ds((i % 8) * 16, 16)] = jnp.zeros((16,), _f32)
        return 0

    lax.fori_loop(0, 32 * 8, z128, 0)

    for p in range(NCHUNK // NC):
        k = p * NC + c
        _s_scatter_pass(lists_hbm, counts_hbm, bvals_hbm, k,
                        list_v, cnt_v, lr0, lr1, co0, co1, r0, r1,
                        zero_v, acc, sem0, sem1)
        lo = k * C
        g0 = lo + s * WB
        pltpu.sync_copy(dv2_hbm.at[pl.ds(g0, WB)], dv2_v.at[pl.ds(0, WB)])
        for q in range((WB + 63) // 64):
            sz = min(64, WB - q * 64)
            pltpu.sync_copy(acc.at[pl.ds(s * WB + q * 64, sz)],
                            sbuf.at[pl.ds(0, sz)])
            if mode == 1:
                pltpu.sync_copy(x0_hbm.at[pl.ds(g0 + q * 64, sz)],
                                xbuf0.at[pl.ds(0, sz)])
                pltpu.sync_copy(x1_hbm.at[pl.ds(g0 + q * 64, sz)],
                                xbuf1.at[pl.ds(0, sz)])

            def wrow(r, _):
                w = dv2_v[pl.ds(q * 64 + r, 16)][0]
                for cc in range(8):
                    srow = sbuf[r, pl.ds(cc * 16, 16)]
                    xv = srow * (w * 0.0625)
                    if mode == 0:
                        obuf1[r, pl.ds(cc * 16, 16)] = xv
                        obuf2[r, pl.ds(cc * 16, 16)] = xv * w
                    else:
                        obuf1[r, pl.ds(cc * 16, 16)] = (
                            xbuf0[r, pl.ds(cc * 16, 16)]
                            + xbuf1[r, pl.ds(cc * 16, 16)] + xv) * (1.0 / 3.0)
                return 0

            lax.fori_loop(0, sz, wrow, 0)
            pltpu.sync_copy(obuf1.at[pl.ds(0, sz)],
                            o1_hbm.at[pl.ds(g0 + q * 64, sz)])
            if mode == 0:
                pltpu.sync_copy(obuf2.at[pl.ds(0, sz)],
                                o2_hbm.at[pl.ds(g0 + q * 64, sz)])
        plsc.subcore_barrier()


def _make_s_kernel(mode):
    body = functools.partial(_s_body, mode)
    if mode == 0:
        def body_wrap(lists_hbm, counts_hbm, bvals_hbm, dv2_hbm,
                      o1_hbm, o2_hbm,
                      list_v, cnt_v, lr0, lr1, co0, co1, r0, r1, zero_v,
                      dv2_v, sbuf, obuf1, obuf2, acc, sem0, sem1):
            return body(lists_hbm, counts_hbm, bvals_hbm, dv2_hbm,
                        None, None, o1_hbm, o2_hbm,
                        list_v, cnt_v, lr0, lr1, co0, co1, r0, r1, zero_v,
                        dv2_v, sbuf, obuf1, obuf2, None, None,
                        acc, sem0, sem1)
        out_type = [jax.ShapeDtypeStruct((NPAD, EMB), _f32)] * 2
    else:
        def body_wrap(lists_hbm, counts_hbm, bvals_hbm, dv2_hbm,
                      x0_hbm, x1_hbm, o1_hbm,
                      list_v, cnt_v, lr0, lr1, co0, co1, r0, r1, zero_v,
                      dv2_v, sbuf, obuf1, xbuf0, xbuf1, acc, sem0, sem1):
            return body(lists_hbm, counts_hbm, bvals_hbm, dv2_hbm,
                        x0_hbm, x1_hbm, o1_hbm, None,
                        list_v, cnt_v, lr0, lr1, co0, co1, r0, r1, zero_v,
                        dv2_v, sbuf, obuf1, None, xbuf0, xbuf1,
                        acc, sem0, sem1)
        out_type = jax.ShapeDtypeStruct((NPAD, EMB), _f32)
    return pl.kernel(
        body_wrap,
        out_type=out_type,
        mesh=_mesh,
        compiler_params=_sc_params,
        scratch_types=[
            pltpu.VMEM((LCAP,), _i32),
            pltpu.VMEM((16,), _i32),
            pltpu.VMEM((1, 128), _i32),
            pltpu.VMEM((1, 128), _i32),
            pltpu.VMEM((1, 128), _i32),
            pltpu.VMEM((1, 128), _i32),
            pltpu.VMEM((128, EMB), _f32),
            pltpu.VMEM((128, EMB), _f32),
            pltpu.VMEM((32, EMB), _f32),
            pltpu.VMEM((WB + 16,), _f32),
            pltpu.VMEM((64, EMB), _f32),
            pltpu.VMEM((64, EMB), _f32),
            pltpu.VMEM((64, EMB), _f32),
            *([pltpu.VMEM((64, EMB), _f32)] if mode == 1 else []),
            pltpu.VMEM_SHARED((NS * ZR, EMB), _f32),
            pltpu.SemaphoreType.DMA,
            pltpu.SemaphoreType.DMA,
        ],
    )


_s1_kernel = _make_s_kernel(0)
_s2_kernel = _make_s_kernel(1)


def _g_body(ir_hbm, b1_hbm, b2_hbm, pos_hbm, neg_hbm, nbsk_hbm,
            posrep, negrep, nb1, nb2,
            idx_v, nidx_v, rows_v, sem):
    c = lax.axis_index("c")
    s = lax.axis_index("s")
    t = c * NS + s
    npt = (B * BSK) // NW

    for which in range(2):
        src = pos_hbm if which == 0 else neg_hbm
        dst = posrep if which == 0 else negrep
        pltpu.sync_copy(src.at[pl.ds(t * npt, npt)], idx_v)

        def batch(i, _):
            pltpu.async_copy(ir_hbm.at[idx_v.at[pl.ds(i * 128, 128)]],
                             rows_v, sem).wait()
            pltpu.sync_copy(rows_v,
                            dst.at[pl.ds(t * npt + i * 128, 128)])
            return 0

        lax.fori_loop(0, npt // 128, batch, 0)

    nbp = B // NW
    pltpu.sync_copy(nbsk_hbm.at[pl.ds(t * nbp, nbp)], nidx_v)
    for which in range(2):
        src = b1_hbm if which == 0 else b2_hbm
        dst = nb1 if which == 0 else nb2
        pltpu.async_copy(src.at[nidx_v], rows_v.at[pl.ds(0, nbp)],
                         sem).wait()
        pltpu.sync_copy(rows_v.at[pl.ds(0, nbp)],
                        dst.at[pl.ds(t * nbp, nbp)])


_g_kernel = pl.kernel(
    _g_body,
    out_type=[
        jax.ShapeDtypeStruct((B * BSK, EMB), _f32),
        jax.ShapeDtypeStruct((B * BSK, EMB), _f32),
        jax.ShapeDtypeStruct((B, EMB), _f32),
        jax.ShapeDtypeStruct((B, EMB), _f32),
    ],
    mesh=_mesh,
    compiler_params=_sc_params,
    scratch_types=[
        pltpu.VMEM(((B * BSK) // NW,), _i32),
        pltpu.VMEM((B // NW,), _i32),
        pltpu.VMEM((128, EMB), _f32),
        pltpu.SemaphoreType.DMA,
    ],
)


def _k8_body(b1_ref, b2_ref, pos_ref, neg_ref, nb1_ref, nb2_ref, l2_ref,
             out_ref, accs):
    i = pl.program_id(0)
    ub = 128

    @pl.when(i == 0)
    def _():
        accs[0] = 0.0
        accs[1] = 0.0

    br = (b1_ref[...] + b2_ref[...]) * 0.125
    br3 = br.reshape(ub, NB, EMB)
    user7 = jnp.sum(br3[:, : NB - 1, :], axis=1) * (1.0 / (NB - 1))
    posb = br3[:, NB - 1, :]
    negb = (nb1_ref[...] + nb2_ref[...]) * 0.125

    eps = 1e-08
    y_ui = jnp.sum(user7 * posb, axis=-1)
    y_uj = jnp.sum(user7 * negb, axis=-1)
    r = jax.nn.sigmoid(y_ui - y_uj)
    r = jnp.where(r == 0.0, eps, r)
    accs[0] += jnp.sum(jnp.log(r))

    pos3 = pos_ref[...].reshape(ub, BSK, EMB)
    neg3 = neg_ref[...].reshape(ub, BSK, EMB)
    yui = jnp.sum(user7[:, None, :] * pos3, axis=-1)
    yuj = jnp.sum(user7[:, None, :] * neg3, axis=-1)
    r2 = jax.nn.sigmoid(yui - yuj)
    r2 = jnp.where(r2 == 0.0, eps, r2)
    accs[1] += jnp.sum(jnp.mean(jnp.log(r2), axis=1))

    lane = lax.broadcasted_iota(_i32, (1, 128), 1)
    loss1 = -accs[0] / B
    loss2 = -accs[1] / B
    l2reg = 0.0001 * (l2_ref[0, 0] / 2.0)
    out_ref[...] = jnp.where(lane == 0, loss1,
                             jnp.where(lane == 1, loss2, l2reg))


def _k8(b1, b2, posrep, negrep, nb1, nb2, l2):
    ub = 128
    return pl.pallas_call(
        _k8_body,
        grid=(B // ub,),
        in_specs=[
            pl.BlockSpec((ub * NB, EMB), lambda i: (i, 0)),
            pl.BlockSpec((ub * NB, EMB), lambda i: (i, 0)),
            pl.BlockSpec((ub * BSK, EMB), lambda i: (i, 0)),
            pl.BlockSpec((ub * BSK, EMB), lambda i: (i, 0)),
            pl.BlockSpec((ub, EMB), lambda i: (i, 0)),
            pl.BlockSpec((ub, EMB), lambda i: (i, 0)),
            pl.BlockSpec(memory_space=pltpu.SMEM),
        ],
        out_specs=pl.BlockSpec((1, 128), lambda i: (0, 0)),
        out_shape=jax.ShapeDtypeStruct((1, 128), _f32),
        scratch_shapes=[pltpu.SMEM((2,), _f32)],
    )(b1, b2, posrep, negrep, nb1, nb2, l2)


def kernel(log_seqs, batch_user_list, item_emb, neg_bsk_idx, neg_items):
    rows = log_seqs.reshape(-1).astype(_i32)
    x0p = jnp.concatenate(
        [item_emb, jnp.zeros((NPAD - N, EMB), _f32)], axis=0)

    dvp, lists, counts = _k1(rows)
    dv2_flat = _k2a(dvp)
    z0, l2 = _k2b(x0p, dv2_flat)

    b1 = _t_kernel(z0, rows)
    x1, z1 = _s1_kernel(lists, counts, b1, dv2_flat)
    b2 = _t_kernel(z1, rows)
    ir = _s2_kernel(lists, counts, b2, dv2_flat, x0p, x1)

    pos_idx = log_seqs[:, NB - 1].reshape(-1).astype(_i32)
    neg_idx = neg_items.reshape(-1).astype(_i32)
    posrep, negrep, nb1, nb2 = _g_kernel(ir, b1, b2, pos_idx, neg_idx,
                                         neg_bsk_idx.astype(_i32))

    out = _k8(b1, b2, posrep, negrep, nb1, nb2, l2)
    return (out[0, 0], out[0, 1], out[0, 2])


# speedup vs baseline: 7.6062x; 2.4909x over previous
import functools

import jax
import jax.numpy as jnp
from jax import lax
from jax.experimental import pallas as pl
from jax.experimental.pallas import tpu as pltpu
from jax.experimental.pallas import tpu_sc as plsc

NUM_ITEM = 50000
EMB = 128
PAD = 50000
B = 1024
NB = 8
BSK = 16
E = B * NB
NE = E * BSK
N = NUM_ITEM + 1
NPAD = 50176
NCHUNK = 8
C = NPAD // NCHUNK
CACC = C + 128
LCAP = 4096 + 128
NC = 2
NS = 16
NW = NC * NS
PER_TILE = NE // NW
HSLC = NPAD // NS
WB = C // NS
ZR = CACC // NS

_mesh = plsc.VectorSubcoreMesh(core_axis_name="c", subcore_axis_name="s")
_sc_params = pltpu.CompilerParams(needs_layout_passes=False)
_f32 = jnp.float32
_i32 = jnp.int32


def _zero_vmem(ref, n):
    z = jnp.zeros((16,), ref.dtype)

    def body(i, _):
        ref[pl.ds(i * 16, 16)] = z
        return 0

    lax.fori_loop(0, n // 16, body, 0)


def _k1_body(rows_hbm, dvp_hbm, lists_hbm, counts_hbm,
             idx_v, hist, b0, b1, b2, b3, b4, b5, b6, b7, cnt_v, sem):
    c = lax.axis_index("c")
    s = lax.axis_index("s")
    t = c * NS + s
    bufs = (b0, b1, b2, b3, b4, b5, b6, b7)

    pltpu.sync_copy(rows_hbm.at[pl.ds(t * PER_TILE, PER_TILE)], idx_v)
    _zero_vmem(hist, NPAD)

    ones16 = jnp.ones((16,), _f32)

    def scan(g, ms):
        idx16 = idx_v[pl.ds(g * 16, 16)]
        plsc.addupdate_scatter(hist, [idx16], ones16)
        basket = t * (PER_TILE // 16) + g
        packed = idx16 * 8192 + basket
        chunk = idx16 // C
        new_ms = []
        for k in range(NCHUNK):
            mask = chunk == k
            plsc.store_compressed(bufs[k].at[pl.ds(ms[k], 16)], packed,
                                  mask=mask)
            new_ms.append(ms[k] + jnp.sum(mask.astype(_i32)))
        return tuple(new_ms)

    ms = lax.fori_loop(0, PER_TILE // 16, scan, (0,) * NCHUNK)

    lane = lax.iota(_i32, 16)
    mp = []
    for k in range(NCHUNK):
        for j in range(8):
            trow = (k + 1) * C + lax.rem(t * 8 + j, 128)
            tcol = t * 128 + j * 16
            bufs[k][pl.ds(ms[k] + j * 16, 16)] = (trow * 8192 + tcol) + lane
        mp.append(((ms[k] + 127) // 128) * 128)

    cvec = jnp.full((16,), 0, _i32)
    for k in range(NCHUNK):
        cvec = jnp.where(lane == k, mp[k], cvec)
    cnt_v[...] = cvec
    pltpu.sync_copy(cnt_v, counts_hbm.at[pl.ds(t * 16, 16)])
    for k in range(NCHUNK):
        pltpu.sync_copy(bufs[k],
                        lists_hbm.at[pl.ds((t * NCHUNK + k) * LCAP, LCAP)])

    pltpu.sync_copy(hist, dvp_hbm.at[pl.ds(t * NPAD, NPAD)])


_k1 = pl.kernel(
    _k1_body,
    out_type=[
        jax.ShapeDtypeStruct((NW * NPAD,), _f32),
        jax.ShapeDtypeStruct((NW * NCHUNK * LCAP,), _i32),
        jax.ShapeDtypeStruct((NW * 16,), _i32),
    ],
    mesh=_mesh,
    compiler_params=_sc_params,
    scratch_types=[
        pltpu.VMEM((PER_TILE,), _i32),
        pltpu.VMEM((NPAD,), _f32),
        *[pltpu.VMEM((LCAP,), _i32) for _ in range(NCHUNK)],
        pltpu.VMEM((16,), _i32),
        pltpu.SemaphoreType.DMA,
    ],
)


def _k2a_body(*refs):
    dv2_ref = refs[-1]
    dv = refs[0][...]
    for r in refs[1:-1]:
        dv = dv + r[...]
    dv2_ref[...] = jnp.where(dv > 0, lax.rsqrt(jnp.maximum(dv, 1.0)), 0.0)


def _k2a(dvp_flat):
    nblk = NPAD // 1024
    return pl.pallas_call(
        _k2a_body,
        grid=(nblk,),
        in_specs=[
            pl.BlockSpec((1024,), lambda i, t=t: (i + t * nblk,))
            for t in range(NW)
        ],
        out_specs=pl.BlockSpec((1024,), lambda i: (i,)),
        out_shape=jax.ShapeDtypeStruct((NPAD,), _f32),
    )(*([dvp_flat] * NW))


def _k2b_body(x0_ref, dv2_ref, z0_ref, l2_ref):
    i = pl.program_id(0)
    x0 = x0_ref[...]
    z0_ref[...] = x0 * dv2_ref[...].reshape(128, 1)

    @pl.when(i == 0)
    def _():
        l2_ref[0, 0] = 0.0

    l2_ref[0, 0] += jnp.sum(x0 * x0)


def _k2b(x0p, dv2_flat):
    return pl.pallas_call(
        _k2b_body,
        grid=(NPAD // 128,),
        in_specs=[
            pl.BlockSpec((128, EMB), lambda i: (i, 0)),
            pl.BlockSpec((128,), lambda i: (i,)),
        ],
        out_specs=[
            pl.BlockSpec((128, EMB), lambda i: (i, 0)),
            pl.BlockSpec(memory_space=pltpu.SMEM),
        ],
        out_shape=[
            jax.ShapeDtypeStruct((NPAD, EMB), _f32),
            jax.ShapeDtypeStruct((1, 1), _f32),
        ],
    )(x0p, dv2_flat)


def _t_body(z_hbm, rows_hbm, b_hbm, idx_v, r0, r1, bidx, zero_v, acc_sh,
            sem0, sem1):
    c = lax.axis_index("c")
    s = lax.axis_index("s")
    t = c * NS + s
    nbk = E // NW
    pltpu.sync_copy(rows_hbm.at[pl.ds(t * PER_TILE, PER_TILE)], idx_v)

    def zv(i, _):
        zero_v[i // 8, pl.ds((i % 8) * 16, 16)] = jnp.zeros((16,), _f32)
        return 0

    lax.fori_loop(0, 32 * 8, zv, 0)
    for q in range(nbk // 32):
        pltpu.sync_copy(zero_v, acc_sh.at[pl.ds(s * nbk + q * 32, 32)])

    nb = PER_TILE // 128
    bufs = (r0, r1)
    sems = (sem0, sem1)
    cps = {0: pltpu.async_copy(z_hbm.at[idx_v.at[pl.ds(0, 128)]], r0, sem0)}
    for j in range(nb):
        if j + 1 < nb:
            cps[j + 1] = pltpu.async_copy(
                z_hbm.at[idx_v.at[pl.ds((j + 1) * 128, 128)]],
                bufs[(j + 1) % 2], sems[(j + 1) % 2])
        cps[j].wait()
        base = s * nbk + j * 8
        for q in range(8):
            bidx[0, pl.ds(q * 16, 16)] = jnp.full((16,), 0, _i32) + (base + q)
        pltpu.sync_copy(bufs[j % 2], acc_sh.at[bidx.at[0]], add=True)

    pltpu.sync_copy(acc_sh.at[pl.ds(s * nbk, nbk)],
                    b_hbm.at[pl.ds(t * nbk, nbk)])


_t_kernel = pl.kernel(
    _t_body,
    out_type=jax.ShapeDtypeStruct((E, EMB), _f32),
    mesh=_mesh,
    compiler_params=_sc_params,
    scratch_types=[
        pltpu.VMEM((PER_TILE,), _i32),
        pltpu.VMEM((128, EMB), _f32),
        pltpu.VMEM((128, EMB), _f32),
        pltpu.VMEM((1, 128), _i32),
        pltpu.VMEM((32, EMB), _f32),
        pltpu.VMEM_SHARED((NS * (E // NW), EMB), _f32),
        pltpu.SemaphoreType.DMA,
        pltpu.SemaphoreType.DMA,
    ],
)


def _s_scatter_pass(lists_hbm, counts_hbm, bvals_hbm, k,
                    list_v, cnt_v, lr0, lr1, co0, co1, r0, r1,
                    zero_v, acc, sem0, sem1):
    s = lax.axis_index("s")
    lo = k * C

    for q in range((ZR + 31) // 32):
        sz = min(32, ZR - q * 32)
        pltpu.sync_copy(zero_v.at[pl.ds(0, sz)],
                        acc.at[pl.ds(s * ZR + q * 32, sz)])
    plsc.subcore_barrier()

    lrs = (lr0, lr1)
    cos = (co0, co1)
    rbs = (r0, r1)
    sems = (sem0, sem1)

    def prep_fire(b, slot):
        for tt in range(8):
            v16 = list_v[pl.ds(b * 128 + tt * 16, 16)]
            cos[slot][0, pl.ds(tt * 16, 16)] = jnp.bitwise_and(v16, 8191)
            lrs[slot][0, pl.ds(tt * 16, 16)] = (
                lax.shift_right_logical(v16, 13) - lo)
        pltpu.async_copy(bvals_hbm.at[cos[slot].at[0]], rbs[slot],
                         sems[slot])

    for rr in range(2):
        r = s * 2 + rr
        pltpu.sync_copy(counts_hbm.at[pl.ds(r * 16, 16)], cnt_v)
        pltpu.sync_copy(lists_hbm.at[pl.ds((r * NCHUNK + k) * LCAP, LCAP)],
                        list_v)
        cv = cnt_v[...]
        nb = jnp.sum(jnp.where(lax.iota(_i32, 16) == k, cv, 0)) // 128

        @pl.when(nb > 0)
        def _():
            prep_fire(0, 0)

        @pl.when(nb > 1)
        def _():
            prep_fire(1, 1)

        def super_(i, _):
            for slot in range(2):
                b = i * 2 + slot

                @pl.when(b < nb)
                def _():
                    pltpu.make_async_copy(bvals_hbm.at[cos[slot].at[0]],
                                          rbs[slot], sems[slot]).wait()
                    pltpu.sync_copy(rbs[slot], acc.at[lrs[slot].at[0]],
                                    add=True)

                    @pl.when(b + 2 < nb)
                    def _():
                        prep_fire(b + 2, slot)
            return 0

        lax.fori_loop(0, (nb + 1) // 2, super_, 0)
    plsc.subcore_barrier()


def _s_body(mode, lists_hbm, counts_hbm, bvals_hbm, dv2_hbm, x0_hbm, x1_hbm,
            o1_hbm, o2_hbm,
            list_v, cnt_v, lr0, lr1, co0, co1, r0, r1, zero_v, dv2_v,
            sbuf, obuf1, obuf2, xbuf0, xbuf1, acc, sem0, sem1):
    c = lax.axis_index("c")
    s = lax.axis_index("s")

    def z128(i, _):
        zero_v[i // 8, pl.ds((i % 8) * 16, 16)] = jnp.zeros((16,), _f32)
        return 0

    lax.fori_loop(0, 32 * 8, z128, 0)

    for p in range(NCHUNK // NC):
        k = p * NC + c
        _s_scatter_pass(lists_hbm, counts_hbm, bvals_hbm, k,
                        list_v, cnt_v, lr0, lr1, co0, co1, r0, r1,
                        zero_v, acc, sem0, sem1)
        lo = k * C
        g0 = lo + s * WB
        pltpu.sync_copy(dv2_hbm.at[pl.ds(g0, WB)], dv2_v.at[pl.ds(0, WB)])
        for q in range((WB + 63) // 64):
            sz = min(64, WB - q * 64)
            pltpu.sync_copy(acc.at[pl.ds(s * WB + q * 64, sz)],
                            sbuf.at[pl.ds(0, sz)])
            if mode == 1:
                pltpu.sync_copy(x0_hbm.at[pl.ds(g0 + q * 64, sz)],
                                xbuf0.at[pl.ds(0, sz)])
                pltpu.sync_copy(x1_hbm.at[pl.ds(g0 + q * 64, sz)],
                                xbuf1.at[pl.ds(0, sz)])

            def wrow(r, _):
                w = dv2_v[pl.ds(q * 64 + r, 16)][0]
                for cc in range(8):
                    srow = sbuf[r, pl.ds(cc * 16, 16)]
                    xv = srow * (w * 0.0625)
                    if mode == 0:
                        obuf1[r, pl.ds(cc * 16, 16)] = xv
                        obuf2[r, pl.ds(cc * 16, 16)] = xv * w
                    else:
                        obuf1[r, pl.ds(cc * 16, 16)] = (
                            xbuf0[r, pl.ds(cc * 16, 16)]
                            + xbuf1[r, pl.ds(cc * 16, 16)] + xv) * (1.0 / 3.0)
                return 0

            lax.fori_loop(0, sz, wrow, 0)
            pltpu.sync_copy(obuf1.at[pl.ds(0, sz)],
                            o1_hbm.at[pl.ds(g0 + q * 64, sz)])
            if mode == 0:
                pltpu.sync_copy(obuf2.at[pl.ds(0, sz)],
                                o2_hbm.at[pl.ds(g0 + q * 64, sz)])
        plsc.subcore_barrier()


def _make_s_kernel(mode):
    body = functools.partial(_s_body, mode)
    if mode == 0:
        def body_wrap(lists_hbm, counts_hbm, bvals_hbm, dv2_hbm,
                      o1_hbm, o2_hbm,
                      list_v, cnt_v, lr0, lr1, co0, co1, r0, r1, zero_v,
                      dv2_v, sbuf, obuf1, obuf2, acc, sem0, sem1):
            return body(lists_hbm, counts_hbm, bvals_hbm, dv2_hbm,
                        None, None, o1_hbm, o2_hbm,
                        list_v, cnt_v, lr0, lr1, co0, co1, r0, r1, zero_v,
                        dv2_v, sbuf, obuf1, obuf2, None, None,
                        acc, sem0, sem1)
        out_type = [jax.ShapeDtypeStruct((NPAD, EMB), _f32)] * 2
    else:
        def body_wrap(lists_hbm, counts_hbm, bvals_hbm, dv2_hbm,
                      x0_hbm, x1_hbm, o1_hbm,
                      list_v, cnt_v, lr0, lr1, co0, co1, r0, r1, zero_v,
                      dv2_v, sbuf, obuf1, xbuf0, xbuf1, acc, sem0, sem1):
            return body(lists_hbm, counts_hbm, bvals_hbm, dv2_hbm,
                        x0_hbm, x1_hbm, o1_hbm, None,
                        list_v, cnt_v, lr0, lr1, co0, co1, r0, r1, zero_v,
                        dv2_v, sbuf, obuf1, None, xbuf0, xbuf1,
                        acc, sem0, sem1)
        out_type = jax.ShapeDtypeStruct((NPAD, EMB), _f32)
    return pl.kernel(
        body_wrap,
        out_type=out_type,
        mesh=_mesh,
        compiler_params=_sc_params,
        scratch_types=[
            pltpu.VMEM((LCAP,), _i32),
            pltpu.VMEM((16,), _i32),
            pltpu.VMEM((1, 128), _i32),
            pltpu.VMEM((1, 128), _i32),
            pltpu.VMEM((1, 128), _i32),
            pltpu.VMEM((1, 128), _i32),
            pltpu.VMEM((128, EMB), _f32),
            pltpu.VMEM((128, EMB), _f32),
            pltpu.VMEM((32, EMB), _f32),
            pltpu.VMEM((WB + 16,), _f32),
            pltpu.VMEM((64, EMB), _f32),
            pltpu.VMEM((64, EMB), _f32),
            pltpu.VMEM((64, EMB), _f32),
            *([pltpu.VMEM((64, EMB), _f32)] if mode == 1 else []),
            pltpu.VMEM_SHARED((NS * ZR, EMB), _f32),
            pltpu.SemaphoreType.DMA,
            pltpu.SemaphoreType.DMA,
        ],
    )


_s1_kernel = _make_s_kernel(0)
_s2_kernel = _make_s_kernel(1)


def _g_body(ir_hbm, b1_hbm, b2_hbm, pos_hbm, neg_hbm, nbsk_hbm,
            posrep, negrep, nb1, nb2,
            idx_v, nidx_v, rows_v, sem):
    c = lax.axis_index("c")
    s = lax.axis_index("s")
    t = c * NS + s
    npt = (B * BSK) // NW

    for which in range(2):
        src = pos_hbm if which == 0 else neg_hbm
        dst = posrep if which == 0 else negrep
        pltpu.sync_copy(src.at[pl.ds(t * npt, npt)], idx_v)

        def batch(i, _):
            pltpu.async_copy(ir_hbm.at[idx_v.at[pl.ds(i * 128, 128)]],
                             rows_v, sem).wait()
            pltpu.sync_copy(rows_v,
                            dst.at[pl.ds(t * npt + i * 128, 128)])
            return 0

        lax.fori_loop(0, npt // 128, batch, 0)

    nbp = B // NW
    pltpu.sync_copy(nbsk_hbm.at[pl.ds(t * nbp, nbp)], nidx_v)
    for which in range(2):
        src = b1_hbm if which == 0 else b2_hbm
        dst = nb1 if which == 0 else nb2
        pltpu.async_copy(src.at[nidx_v], rows_v.at[pl.ds(0, nbp)],
                         sem).wait()
        pltpu.sync_copy(rows_v.at[pl.ds(0, nbp)],
                        dst.at[pl.ds(t * nbp, nbp)])


_g_kernel = pl.kernel(
    _g_body,
    out_type=[
        jax.ShapeDtypeStruct((B * BSK, EMB), _f32),
        jax.ShapeDtypeStruct((B * BSK, EMB), _f32),
        jax.ShapeDtypeStruct((B, EMB), _f32),
        jax.ShapeDtypeStruct((B, EMB), _f32),
    ],
    mesh=_mesh,
    compiler_params=_sc_params,
    scratch_types=[
        pltpu.VMEM(((B * BSK) // NW,), _i32),
        pltpu.VMEM((B // NW,), _i32),
        pltpu.VMEM((128, EMB), _f32),
        pltpu.SemaphoreType.DMA,
    ],
)


def _k8_body(b1_ref, b2_ref, pos_ref, neg_ref, nb1_ref, nb2_ref, l2_ref,
             out_ref, accs):
    i = pl.program_id(0)
    ub = 128

    @pl.when(i == 0)
    def _():
        accs[0] = 0.0
        accs[1] = 0.0

    br = (b1_ref[...] + b2_ref[...]) * 0.125
    br3 = br.reshape(ub, NB, EMB)
    user7 = jnp.sum(br3[:, : NB - 1, :], axis=1) * (1.0 / (NB - 1))
    posb = br3[:, NB - 1, :]
    negb = (nb1_ref[...] + nb2_ref[...]) * 0.125

    eps = 1e-08
    y_ui = jnp.sum(user7 * posb, axis=-1)
    y_uj = jnp.sum(user7 * negb, axis=-1)
    r = jax.nn.sigmoid(y_ui - y_uj)
    r = jnp.where(r == 0.0, eps, r)
    accs[0] += jnp.sum(jnp.log(r))

    pos3 = pos_ref[...].reshape(ub, BSK, EMB)
    neg3 = neg_ref[...].reshape(ub, BSK, EMB)
    yui = jnp.sum(user7[:, None, :] * pos3, axis=-1)
    yuj = jnp.sum(user7[:, None, :] * neg3, axis=-1)
    r2 = jax.nn.sigmoid(yui - yuj)
    r2 = jnp.where(r2 == 0.0, eps, r2)
    accs[1] += jnp.sum(jnp.mean(jnp.log(r2), axis=1))

    lane = lax.broadcasted_iota(_i32, (1, 128), 1)
    loss1 = -accs[0] / B
    loss2 = -accs[1] / B
    l2reg = 0.0001 * (l2_ref[0, 0] / 2.0)
    out_ref[...] = jnp.where(lane == 0, loss1,
                             jnp.where(lane == 1, loss2, l2reg))


def _k8(b1, b2, posrep, negrep, nb1, nb2, l2):
    ub = 128
    return pl.pallas_call(
        _k8_body,
        grid=(B // ub,),
        in_specs=[
            pl.BlockSpec((ub * NB, EMB), lambda i: (i, 0)),
            pl.BlockSpec((ub * NB, EMB), lambda i: (i, 0)),
            pl.BlockSpec((ub * BSK, EMB), lambda i: (i, 0)),
            pl.BlockSpec((ub * BSK, EMB), lambda i: (i, 0)),
            pl.BlockSpec((ub, EMB), lambda i: (i, 0)),
            pl.BlockSpec((ub, EMB), lambda i: (i, 0)),
            pl.BlockSpec(memory_space=pltpu.SMEM),
        ],
        out_specs=pl.BlockSpec((1, 128), lambda i: (0, 0)),
        out_shape=jax.ShapeDtypeStruct((1, 128), _f32),
        scratch_shapes=[pltpu.SMEM((2,), _f32)],
    )(b1, b2, posrep, negrep, nb1, nb2, l2)


def kernel(log_seqs, batch_user_list, item_emb, neg_bsk_idx, neg_items):
    rows = log_seqs.reshape(-1).astype(_i32)
    x0p = jnp.concatenate(
        [item_emb, jnp.zeros((NPAD - N, EMB), _f32)], axis=0)

    dvp, lists, counts = _k1(rows)
    dv2_flat = _k2a(dvp)
    z0, l2 = _k2b(x0p, dv2_flat)

    b1 = _t_kernel(z0, rows)
    x1, z1 = _s1_kernel(lists, counts, b1, dv2_flat)
    b2 = _t_kernel(z1, rows)
    ir = _s2_kernel(lists, counts, b2, dv2_flat, x0p, x1)

    pos_idx = log_seqs[:, NB - 1].reshape(-1).astype(_i32)
    neg_idx = neg_items.reshape(-1).astype(_i32)
    posrep, negrep, nb1, nb2 = _g_kernel(ir, b1, b2, pos_idx, neg_idx,
                                         neg_bsk_idx.astype(_i32))

    out = _k8(b1, b2, posrep, negrep, nb1, nb2, l2)
    return (out[0, 0], out[0, 1], out[0, 2])
